# Initial kernel scaffold; baseline (speedup 1.0000x reference)
import functools

import jax
import jax.numpy as jnp
from jax.experimental import pallas as pl

NUM_HIDDENS = 384
EMBED_DIM = 256
N_EMBED = 1024
KL_WEIGHT = 0.0005
B = 8
HW = 576


def _tc_body(z_ref, w_ref, b_ref, et_ref, ind_ref, zq_ref, loss_ref):
    b = pl.program_id(0)
    zb = z_ref[0]
    logits = jax.lax.dot_general(
        w_ref[...], zb, (((1,), (0,)), ((), ())),
        preferred_element_type=jnp.float32)
    logits = logits + b_ref[...]
    m = jnp.max(logits, axis=0, keepdims=True)
    e = jnp.exp(logits - m)
    zsum = jnp.sum(e, axis=0, keepdims=True)
    qy = e / zsum
    mq = jnp.max(qy, axis=0, keepdims=True)
    rows = jax.lax.broadcasted_iota(jnp.int32, (N_EMBED, HW), 0)
    ind = jnp.min(jnp.where(qy == mq, rows, jnp.int32(1 << 30)), axis=0)
    ind_ref[0, 0, :] = ind
    onehot = (rows == ind[None, :]).astype(jnp.float32)
    zq = jax.lax.dot_general(
        et_ref[...], onehot, (((1,), (0,)), ((), ())),
        preferred_element_type=jnp.float32)
    zq_ref[0] = zq
    kl = jnp.sum(qy * jnp.log(qy * N_EMBED + 1e-10))

    @pl.when(b == 0)
    def _():
        loss_ref[0, 0] = 0.0

    loss_ref[0, 0] += kl

    @pl.when(b == B - 1)
    def _():
        loss_ref[0, 0] *= jnp.float32(KL_WEIGHT / (B * HW))


@functools.partial(jax.jit, static_argnames=("interpret",))
def kernel(z, W_proj, b_proj, embed_w, interpret=False):
    zf = z.reshape(B, NUM_HIDDENS, HW)
    b2 = b_proj.reshape(N_EMBED, 1)
    embed_wT = embed_w.T
    ind3, zq, loss = pl.pallas_call(
        _tc_body,
        grid=(B,),
        in_specs=[
            pl.BlockSpec((1, NUM_HIDDENS, HW), lambda b: (b, 0, 0)),
            pl.BlockSpec((N_EMBED, NUM_HIDDENS), lambda b: (0, 0)),
            pl.BlockSpec((N_EMBED, 1), lambda b: (0, 0)),
            pl.BlockSpec((EMBED_DIM, N_EMBED), lambda b: (0, 0)),
        ],
        out_specs=[
            pl.BlockSpec((1, 1, HW), lambda b: (b, 0, 0)),
            pl.BlockSpec((1, EMBED_DIM, HW), lambda b: (b, 0, 0)),
            pl.BlockSpec((1, 1), lambda b: (0, 0)),
        ],
        out_shape=[
            jax.ShapeDtypeStruct((B, 1, HW), jnp.int32),
            jax.ShapeDtypeStruct((B, EMBED_DIM, HW), jnp.float32),
            jax.ShapeDtypeStruct((1, 1), jnp.float32),
        ],
        interpret=interpret,
    )(zf, W_proj, b2, embed_wT)
    z_q = zq.reshape(B, EMBED_DIM, 24, 24)
    ind = ind3.reshape(B, 24, 24)
    prior_loss = loss[0, 0]
    return (z_q, prior_loss, ind)


# speedup vs baseline: 1.3326x; 1.3326x over previous
import functools

import jax
import jax.numpy as jnp
from jax.experimental import pallas as pl

NUM_HIDDENS = 384
EMBED_DIM = 256
N_EMBED = 1024
KL_WEIGHT = 0.0005
B = 8
HW = 576


def _tc_body(z_ref, w_ref, b_ref, et_ref, ind_ref, zq_ref, loss_ref):
    b = pl.program_id(0)
    zb = z_ref[0]
    logits = jax.lax.dot_general(
        w_ref[...], zb, (((1,), (0,)), ((), ())),
        preferred_element_type=jnp.float32)
    logits = logits + b_ref[...]
    m = jnp.max(logits, axis=0, keepdims=True)
    e = jnp.exp(logits - m)
    zsum = jnp.sum(e, axis=0, keepdims=True)
    qy = e / zsum
    mq = jnp.max(qy, axis=0, keepdims=True)
    rows = jax.lax.broadcasted_iota(jnp.int32, (N_EMBED, HW), 0)
    ind = jnp.min(jnp.where(qy == mq, rows, jnp.int32(1 << 30)), axis=0)
    ind_ref[0, 0, :] = ind
    onehot = (rows == ind[None, :]).astype(jnp.float32)
    zq = jax.lax.dot_general(
        et_ref[...], onehot, (((1,), (0,)), ((), ())),
        preferred_element_type=jnp.float32)
    zq_ref[0] = zq
    kl = jnp.sum(qy * jnp.log(qy * N_EMBED + 1e-10))

    @pl.when(b == 0)
    def _():
        loss_ref[...] = jnp.zeros((1, 1), jnp.float32)

    loss_ref[...] += jnp.full((1, 1), kl, jnp.float32)

    @pl.when(b == B - 1)
    def _():
        loss_ref[...] *= jnp.float32(KL_WEIGHT / (B * HW))


@functools.partial(jax.jit, static_argnames=("interpret",))
def kernel(z, W_proj, b_proj, embed_w, interpret=False):
    zf = z.reshape(B, NUM_HIDDENS, HW)
    b2 = b_proj.reshape(N_EMBED, 1)
    embed_wT = embed_w.T
    ind3, zq, loss = pl.pallas_call(
        _tc_body,
        grid=(B,),
        in_specs=[
            pl.BlockSpec((1, NUM_HIDDENS, HW), lambda b: (b, 0, 0)),
            pl.BlockSpec((N_EMBED, NUM_HIDDENS), lambda b: (0, 0)),
            pl.BlockSpec((N_EMBED, 1), lambda b: (0, 0)),
            pl.BlockSpec((EMBED_DIM, N_EMBED), lambda b: (0, 0)),
        ],
        out_specs=[
            pl.BlockSpec((1, 1, HW), lambda b: (b, 0, 0)),
            pl.BlockSpec((1, EMBED_DIM, HW), lambda b: (b, 0, 0)),
            pl.BlockSpec((1, 1), lambda b: (0, 0)),
        ],
        out_shape=[
            jax.ShapeDtypeStruct((B, 1, HW), jnp.int32),
            jax.ShapeDtypeStruct((B, EMBED_DIM, HW), jnp.float32),
            jax.ShapeDtypeStruct((1, 1), jnp.float32),
        ],
        interpret=interpret,
    )(zf, W_proj, b2, embed_wT)
    z_q = zq.reshape(B, EMBED_DIM, 24, 24)
    ind = ind3.reshape(B, 24, 24)
    prior_loss = loss[0, 0]
    return (z_q, prior_loss, ind)
